# baseline (speedup 1.0000x reference)
import functools

import jax
import jax.numpy as jnp
from jax import lax
from jax.experimental import pallas as pl
from jax.experimental.pallas import tpu as pltpu
from jax.experimental.pallas import tpu_sc as plsc

SC_NC = 2
SC_NS = 16
SC_NW = SC_NC * SC_NS
SC_L = 16

N_TOK = 2048
HID = 768
EDIM = 512
NE = 64
BLK = 64
RPAD = 8192
NBLK = RPAD // BLK


def _gelu_tanh(x):
    return 0.5 * x * (1.0 + jnp.tanh(jnp.sqrt(2.0 / jnp.pi) * (x + 0.044715 * x ** 3)))


def _router_plan_body(x_ref, rs_ref, pes_ref, rw_ref, post_ref, wt_ref,
                      b2e_ref, nb_ref):
    x = x_ref[...]
    xr = x * lax.rsqrt(jnp.mean(x * x, axis=-1, keepdims=True) + 1e-6)
    ri = xr * rs_ref[...] * (HID ** -0.5)
    logits = lax.dot_general(ri, rw_ref[...], (((1,), (1,)), ((), ())),
                             preferred_element_type=jnp.float32)
    m = jnp.max(logits, axis=-1, keepdims=True)
    p = jnp.exp(logits - m)
    probs = p / jnp.sum(p, axis=-1, keepdims=True)

    e_iota = lax.broadcasted_iota(jnp.int32, (N_TOK, NE), 1)
    l1 = jnp.max(logits, axis=-1, keepdims=True)
    i1 = jnp.min(jnp.where(logits == l1, e_iota, NE), axis=-1, keepdims=True)
    masked = jnp.where(e_iota == i1, -jnp.inf, logits)
    l2 = jnp.max(masked, axis=-1, keepdims=True)
    i2 = jnp.min(jnp.where(masked == l2, e_iota, NE), axis=-1, keepdims=True)

    oh1 = (e_iota == i1).astype(jnp.float32)
    oh2 = (e_iota == i2).astype(jnp.float32)
    pes = pes_ref[...]
    p1 = jnp.sum(oh1 * probs, axis=-1, keepdims=True)
    p2 = jnp.sum(oh2 * probs, axis=-1, keepdims=True)
    s = p1 + p2
    w1 = p1 / s * jnp.sum(oh1 * pes, axis=-1, keepdims=True)
    w2 = p2 / s * jnp.sum(oh2 * pes, axis=-1, keepdims=True)

    M = jnp.concatenate([oh1, oh2], axis=0)
    c = M
    sh = 1
    while sh < 2 * N_TOK:
        c = c + jnp.concatenate(
            [jnp.zeros((sh, NE), jnp.float32), c[:2 * N_TOK - sh]], axis=0)
        sh *= 2
    rank = jnp.sum(M * (c - M), axis=-1, keepdims=True)

    counts = jnp.sum(M, axis=0, keepdims=True)
    bpe = jnp.floor((counts + (BLK - 1)) * (1.0 / BLK))
    ut = (lax.broadcasted_iota(jnp.int32, (NE, NE), 0)
          <= lax.broadcasted_iota(jnp.int32, (NE, NE), 1)).astype(jnp.float32)
    cumb = lax.dot_general(bpe, ut, (((1,), (0,)), ((), ())),
                           preferred_element_type=jnp.float32)
    row_off = (cumb - bpe) * float(BLK)
    pos = jnp.sum(M * row_off, axis=-1, keepdims=True) + rank
    post_ref[...] = pos.astype(jnp.int32).reshape(2, N_TOK)
    wt_ref[...] = jnp.concatenate([w1, w2], axis=0).reshape(2, N_TOK)

    b_iota = lax.broadcasted_iota(jnp.int32, (NBLK, NE), 0)
    b2e = jnp.sum((cumb.astype(jnp.int32) <= b_iota).astype(jnp.int32),
                  axis=-1, keepdims=True)
    last_used = jnp.max(jnp.where(counts > 0,
                                  lax.broadcasted_iota(jnp.int32, (1, NE), 1),
                                  0), axis=-1, keepdims=True)
    b2e = jnp.minimum(b2e, last_used)
    b2e_ref[...] = b2e.reshape(1, NBLK)
    nb_ref[...] = cumb[:, NE - 1:].astype(jnp.int32)


def _router_plan(x, router_scale, per_expert_scale, router_w):
    return pl.pallas_call(
        _router_plan_body,
        out_shape=[
            jax.ShapeDtypeStruct((2, N_TOK), jnp.int32),
            jax.ShapeDtypeStruct((2, N_TOK), jnp.float32),
            jax.ShapeDtypeStruct((1, NBLK), jnp.int32),
            jax.ShapeDtypeStruct((1, 1), jnp.int32),
        ],
    )(x, router_scale.reshape(1, HID), per_expert_scale.reshape(1, NE),
      router_w)


_HBLK = NBLK // 2


def _make_mlp_half(off, aliased):
    def body(b2e_ref, nb_ref, xs_ref, gate_ref, up_ref, dna_ref, dnb_ref,
             ws_ref, *rest):
        del b2e_ref
        out_ref = rest[-1]

        @pl.when(pl.program_id(0) + off < nb_ref[0])
        def _():
            xb = xs_ref[...]

            def dot(a, wref):
                return lax.dot_general(a, wref[0, 0],
                                       (((1,), (1,)), ((), ())),
                                       preferred_element_type=jnp.float32)
            h = _gelu_tanh(dot(xb, gate_ref)) * dot(xb, up_ref)
            w = ws_ref[...]
            for i, dref in enumerate((dna_ref, dnb_ref)):
                out_ref[:, i * (HID // 2):(i + 1) * (HID // 2)] = (
                    dot(h, dref) * w)

    def gmap(b, nb):
        if off == 0:
            return jnp.minimum(b, jnp.minimum(nb[0], _HBLK) - 1)
        return jnp.clip(jnp.minimum(b + off, nb[0] - 1), off, NBLK - 1)

    def _xs_map(b, s, nb):
        return (gmap(b, nb) - off, 0)

    def _ws_map(b, s, nb):
        return (gmap(b, nb), 0)

    def _w_map(split):
        def m(b, s, nb):
            return (s[gmap(b, nb)], split, 0, 0)
        return m

    in_specs = [
        pl.BlockSpec((BLK, HID), _xs_map),
        pl.BlockSpec((1, 1, EDIM, HID), _w_map(0)),
        pl.BlockSpec((1, 1, EDIM, HID), _w_map(1)),
        pl.BlockSpec((1, 1, HID // 2, EDIM), _w_map(0)),
        pl.BlockSpec((1, 1, HID // 2, EDIM), _w_map(1)),
        pl.BlockSpec((BLK, 1), _ws_map),
    ]
    kwargs = {}
    if aliased:
        in_specs.append(pl.BlockSpec(memory_space=pl.ANY))
        kwargs["input_output_aliases"] = {8: 0}

    grid_spec = pltpu.PrefetchScalarGridSpec(
        num_scalar_prefetch=2,
        grid=(_HBLK,),
        in_specs=in_specs,
        out_specs=pl.BlockSpec((BLK, HID),
                               lambda b, s, nb: (gmap(b, nb), 0)),
    )

    def call(xs_half, gup4, dn4, ws2d, b2e, nbv, hw_in=None):
        args = [b2e, nbv, xs_half, gup4, gup4, dn4, dn4, ws2d]
        if aliased:
            args.append(hw_in)
        return pl.pallas_call(
            body,
            grid_spec=grid_spec,
            out_shape=jax.ShapeDtypeStruct((RPAD, HID), jnp.float32),
            **kwargs,
        )(*args)
    return call


_mlp_half0 = _make_mlp_half(0, aliased=False)
_mlp_half1 = _make_mlp_half(_HBLK, aliased=True)


def _grouped_mlp(xs_a, xs_b, gate_up_proj, down_proj, w_sorted, b2e, nbv):
    gup4 = gate_up_proj.reshape(NE, 2, EDIM, HID)
    dn4 = down_proj.reshape(NE, 2, HID // 2, EDIM)
    ws2d = w_sorted.reshape(RPAD, 1)
    hw_a = _mlp_half0(xs_a, gup4, dn4, ws2d, b2e, nbv)
    return _mlp_half1(xs_b, gup4, dn4, ws2d, b2e, nbv, hw_in=hw_a)


_SC_MESH = plsc.VectorSubcoreMesh(core_axis_name="c", subcore_axis_name="s")


def _sc_wid():
    return lax.axis_index("s") * SC_NC + lax.axis_index("c")


@functools.partial(
    pl.kernel,
    out_type=[jax.ShapeDtypeStruct((RPAD,), jnp.int32),
              jax.ShapeDtypeStruct((RPAD,), jnp.float32)],
    mesh=_SC_MESH,
    scratch_types=[pltpu.VMEM((2 * N_TOK,), jnp.int32),
                   pltpu.VMEM((2 * N_TOK,), jnp.float32),
                   pltpu.VMEM((RPAD,), jnp.int32),
                   pltpu.VMEM((RPAD,), jnp.float32)],
    compiler_params=pltpu.CompilerParams(needs_layout_passes=False),
)
def _sc_scatter_plan(pos_hbm, w_hbm, ts_hbm, ws_hbm, pos_v, w_v, ts_v, ws_v):
    @pl.when(_sc_wid() == 0)
    def _():
        pltpu.sync_copy(pos_hbm, pos_v)
        pltpu.sync_copy(w_hbm, w_v)

        def init(i, carry):
            ts_v[pl.ds(i * SC_L, SC_L)] = (
                (lax.iota(jnp.int32, SC_L) + i * SC_L) & (N_TOK - 1))
            ws_v[pl.ds(i * SC_L, SC_L)] = jnp.zeros((SC_L,), jnp.float32)
            return carry
        lax.fori_loop(0, RPAD // SC_L, init, 0)

        def scat(i, carry):
            sl = pl.ds(i * SC_L, SC_L)
            idx = pos_v[sl]
            tok = (lax.iota(jnp.int32, SC_L) + i * SC_L) & (N_TOK - 1)
            plsc.store_scatter(ts_v, [idx], tok)
            plsc.store_scatter(ws_v, [idx], w_v[sl])
            return carry
        lax.fori_loop(0, 2 * N_TOK // SC_L, scat, 0)
        pltpu.sync_copy(ts_v, ts_hbm)
        pltpu.sync_copy(ws_v, ws_hbm)


_GCH = 32
_GNB = 4
_RHALF = RPAD // 2
_GPT = _RHALF // SC_NW


def _make_sc_gather_half(half):
    @functools.partial(
        pl.kernel,
        out_type=jax.ShapeDtypeStruct((_RHALF, HID), jnp.float32),
        mesh=_SC_MESH,
        scratch_types=[pltpu.VMEM((_GPT,), jnp.int32),
                       [pltpu.VMEM((_GCH, HID), jnp.float32)] * _GNB,
                       [pltpu.SemaphoreType.DMA] * _GNB,
                       [pltpu.SemaphoreType.DMA] * _GNB],
        name=f"sc_gather_x_{half}",
    )
    def _sc_gather_x(x_hbm, ts_hbm, xs_hbm, idx_v, rows_bufs, gsems, wsems):
        wid = _sc_wid()
        base = wid * _GPT
        pltpu.sync_copy(ts_hbm.at[pl.ds(half * _RHALF + base, _GPT)], idx_v)
        nch = _GPT // _GCH

        def issue_gather(i, b):
            pltpu.async_copy(x_hbm.at[idx_v.at[pl.ds(i * _GCH, _GCH)]],
                             rows_bufs[b], gsems[b])

        for b in range(_GNB):
            issue_gather(b, b)

        def step(i, carry):
            for b in range(_GNB):
                @pl.when(i % _GNB == b)
                def _():
                    pltpu.make_async_copy(
                        x_hbm.at[idx_v.at[pl.ds(0, _GCH)]], rows_bufs[b],
                        gsems[b]).wait()
                    pltpu.async_copy(rows_bufs[b],
                                     xs_hbm.at[pl.ds(base + i * _GCH, _GCH)],
                                     wsems[b])

                    @pl.when(i + _GNB < nch)
                    def _():
                        pltpu.make_async_copy(
                            rows_bufs[b], xs_hbm.at[pl.ds(base, _GCH)],
                            wsems[b]).wait()
                        issue_gather(i + _GNB, b)
            return carry
        lax.fori_loop(0, nch, step, 0)

        for b in range(_GNB):
            pltpu.make_async_copy(rows_bufs[b], xs_hbm.at[pl.ds(0, _GCH)],
                                  wsems[b]).wait()
    return _sc_gather_x


_sc_gather_half0 = _make_sc_gather_half(0)
_sc_gather_half1 = _make_sc_gather_half(1)


_CCH = 32


@functools.partial(
    pl.kernel,
    out_type=jax.ShapeDtypeStruct((N_TOK, HID), jnp.float32),
    mesh=_SC_MESH,
    scratch_types=[pltpu.VMEM((_CCH,), jnp.int32),
                   pltpu.VMEM((_CCH,), jnp.int32),
                   pltpu.VMEM((_CCH, HID), jnp.float32),
                   pltpu.VMEM((_CCH, HID), jnp.float32),
                   pltpu.SemaphoreType.DMA,
                   pltpu.SemaphoreType.DMA],
)
def _sc_combine(hw_hbm, p0_hbm, p1_hbm, out_hbm, i0_v, i1_v, b0_v, b1_v,
                s0, s1):
    wid = _sc_wid()

    def chunk(i, carry):
        base = wid * (N_TOK // SC_NW) + i * _CCH
        pltpu.sync_copy(p0_hbm.at[pl.ds(base, _CCH)], i0_v)
        pltpu.sync_copy(p1_hbm.at[pl.ds(base, _CCH)], i1_v)
        c0 = pltpu.async_copy(hw_hbm.at[i0_v], b0_v, s0)
        c1 = pltpu.async_copy(hw_hbm.at[i1_v], b1_v, s1)
        c0.wait()
        c1.wait()

        def addrow(r, carry2):
            for cc in range(HID // SC_L):
                sl = pl.ds(cc * SC_L, SC_L)
                b0_v[r, sl] = b0_v[r, sl] + b1_v[r, sl]
            return carry2
        lax.fori_loop(0, _CCH, addrow, 0)
        pltpu.sync_copy(b0_v, out_hbm.at[pl.ds(base, _CCH)])
        return carry
    lax.fori_loop(0, N_TOK // SC_NW // _CCH, chunk, 0)


def kernel(x, router_scale, per_expert_scale, router_w, gate_up_proj,
           down_proj):
    pos2, w2, b2e, nb = _router_plan(x, router_scale, per_expert_scale,
                                     router_w)
    pos_flat = pos2.reshape(2 * N_TOK)
    w_flat = w2.reshape(2 * N_TOK)

    tok_sorted, w_sorted = _sc_scatter_plan(pos_flat, w_flat)
    xs_a = _sc_gather_half0(x, tok_sorted)
    xs_b = _sc_gather_half1(x, tok_sorted)
    hw = _grouped_mlp(xs_a, xs_b, gate_up_proj, down_proj, w_sorted,
                      b2e.reshape(NBLK), nb.reshape(1))
    out = _sc_combine(hw, pos2[0], pos2[1])
    return out


# speedup vs baseline: 1.0073x; 1.0073x over previous
import functools

import jax
import jax.numpy as jnp
from jax import lax
from jax.experimental import pallas as pl
from jax.experimental.pallas import tpu as pltpu
from jax.experimental.pallas import tpu_sc as plsc

SC_NC = 2
SC_NS = 16
SC_NW = SC_NC * SC_NS
SC_L = 16

N_TOK = 2048
HID = 768
EDIM = 512
NE = 64
BLK = 64
RPAD = 8192
NBLK = RPAD // BLK
_RSPLIT = 2048


def _gelu_tanh(x):
    return 0.5 * x * (1.0 + jnp.tanh(jnp.sqrt(2.0 / jnp.pi) * (x + 0.044715 * x ** 3)))


def _router_plan_body(x_ref, rs_ref, pes_ref, rw_ref, post_ref, wt_ref,
                      b2e_ref, nb_ref):
    x = x_ref[...]
    xr = x * lax.rsqrt(jnp.mean(x * x, axis=-1, keepdims=True) + 1e-6)
    ri = xr * rs_ref[...] * (HID ** -0.5)
    logits = lax.dot_general(ri, rw_ref[...], (((1,), (1,)), ((), ())),
                             preferred_element_type=jnp.float32)
    m = jnp.max(logits, axis=-1, keepdims=True)
    p = jnp.exp(logits - m)
    probs = p / jnp.sum(p, axis=-1, keepdims=True)

    e_iota = lax.broadcasted_iota(jnp.int32, (N_TOK, NE), 1)
    l1 = jnp.max(logits, axis=-1, keepdims=True)
    i1 = jnp.min(jnp.where(logits == l1, e_iota, NE), axis=-1, keepdims=True)
    masked = jnp.where(e_iota == i1, -jnp.inf, logits)
    l2 = jnp.max(masked, axis=-1, keepdims=True)
    i2 = jnp.min(jnp.where(masked == l2, e_iota, NE), axis=-1, keepdims=True)

    oh1 = (e_iota == i1).astype(jnp.float32)
    oh2 = (e_iota == i2).astype(jnp.float32)
    pes = pes_ref[...]
    p1 = jnp.sum(oh1 * probs, axis=-1, keepdims=True)
    p2 = jnp.sum(oh2 * probs, axis=-1, keepdims=True)
    s = p1 + p2
    w1 = p1 / s * jnp.sum(oh1 * pes, axis=-1, keepdims=True)
    w2 = p2 / s * jnp.sum(oh2 * pes, axis=-1, keepdims=True)

    M = jnp.concatenate([oh1, oh2], axis=0)
    c = M
    sh = 1
    while sh < 2 * N_TOK:
        c = c + jnp.concatenate(
            [jnp.zeros((sh, NE), jnp.float32), c[:2 * N_TOK - sh]], axis=0)
        sh *= 2
    rank = jnp.sum(M * (c - M), axis=-1, keepdims=True)

    counts = jnp.sum(M, axis=0, keepdims=True)
    bpe = jnp.floor((counts + (BLK - 1)) * (1.0 / BLK))
    ut = (lax.broadcasted_iota(jnp.int32, (NE, NE), 0)
          <= lax.broadcasted_iota(jnp.int32, (NE, NE), 1)).astype(jnp.float32)
    cumb = lax.dot_general(bpe, ut, (((1,), (0,)), ((), ())),
                           preferred_element_type=jnp.float32)
    row_off = (cumb - bpe) * float(BLK)
    pos = jnp.sum(M * row_off, axis=-1, keepdims=True) + rank
    post_ref[...] = pos.astype(jnp.int32).reshape(2, N_TOK)
    wt_ref[...] = jnp.concatenate([w1, w2], axis=0).reshape(2, N_TOK)

    b_iota = lax.broadcasted_iota(jnp.int32, (NBLK, NE), 0)
    b2e = jnp.sum((cumb.astype(jnp.int32) <= b_iota).astype(jnp.int32),
                  axis=-1, keepdims=True)
    last_used = jnp.max(jnp.where(counts > 0,
                                  lax.broadcasted_iota(jnp.int32, (1, NE), 1),
                                  0), axis=-1, keepdims=True)
    b2e = jnp.minimum(b2e, last_used)
    b2e_ref[...] = b2e.reshape(1, NBLK)
    nb_ref[...] = cumb[:, NE - 1:].astype(jnp.int32)


def _router_plan(x, router_scale, per_expert_scale, router_w):
    return pl.pallas_call(
        _router_plan_body,
        out_shape=[
            jax.ShapeDtypeStruct((2, N_TOK), jnp.int32),
            jax.ShapeDtypeStruct((2, N_TOK), jnp.float32),
            jax.ShapeDtypeStruct((1, NBLK), jnp.int32),
            jax.ShapeDtypeStruct((1, 1), jnp.int32),
        ],
    )(x, router_scale.reshape(1, HID), per_expert_scale.reshape(1, NE),
      router_w)


_HBLK = _RSPLIT // BLK


def _make_mlp_half(off, ngrid, aliased):
    def body(b2e_ref, nb_ref, xs_ref, gate_ref, up_ref, dna_ref, dnb_ref,
             ws_ref, *rest):
        del b2e_ref
        out_ref = rest[-1]

        @pl.when(pl.program_id(0) + off < nb_ref[0])
        def _():
            xb = xs_ref[...]

            def dot(a, wref):
                return lax.dot_general(a, wref[0, 0],
                                       (((1,), (1,)), ((), ())),
                                       preferred_element_type=jnp.float32)
            h = _gelu_tanh(dot(xb, gate_ref)) * dot(xb, up_ref)
            w = ws_ref[...]
            for i, dref in enumerate((dna_ref, dnb_ref)):
                out_ref[:, i * (HID // 2):(i + 1) * (HID // 2)] = (
                    dot(h, dref) * w)

    def gmap(b, nb):
        if off == 0:
            return jnp.minimum(b, jnp.minimum(nb[0], _HBLK) - 1)
        return jnp.clip(jnp.minimum(b + off, nb[0] - 1), off, NBLK - 1)

    def _xs_map(b, s, nb):
        return (gmap(b, nb) - off, 0)

    def _ws_map(b, s, nb):
        return (gmap(b, nb), 0)

    def _w_map(split):
        def m(b, s, nb):
            return (s[gmap(b, nb)], split, 0, 0)
        return m

    in_specs = [
        pl.BlockSpec((BLK, HID), _xs_map),
        pl.BlockSpec((1, 1, EDIM, HID), _w_map(0)),
        pl.BlockSpec((1, 1, EDIM, HID), _w_map(1)),
        pl.BlockSpec((1, 1, HID // 2, EDIM), _w_map(0)),
        pl.BlockSpec((1, 1, HID // 2, EDIM), _w_map(1)),
        pl.BlockSpec((BLK, 1), _ws_map),
    ]
    kwargs = {}
    if aliased:
        in_specs.append(pl.BlockSpec(memory_space=pl.ANY))
        kwargs["input_output_aliases"] = {8: 0}

    grid_spec = pltpu.PrefetchScalarGridSpec(
        num_scalar_prefetch=2,
        grid=(ngrid,),
        in_specs=in_specs,
        out_specs=pl.BlockSpec((BLK, HID),
                               lambda b, s, nb: (gmap(b, nb), 0)),
    )

    def call(xs_half, gup4, dn4, ws2d, b2e, nbv, hw_in=None):
        args = [b2e, nbv, xs_half, gup4, gup4, dn4, dn4, ws2d]
        if aliased:
            args.append(hw_in)
        return pl.pallas_call(
            body,
            grid_spec=grid_spec,
            out_shape=jax.ShapeDtypeStruct((RPAD, HID), jnp.float32),
            **kwargs,
        )(*args)
    return call


_mlp_half0 = _make_mlp_half(0, _HBLK, aliased=False)
_mlp_half1 = _make_mlp_half(_HBLK, NBLK - _HBLK, aliased=True)


def _grouped_mlp(xs_a, xs_b, gate_up_proj, down_proj, w_sorted, b2e, nbv):
    gup4 = gate_up_proj.reshape(NE, 2, EDIM, HID)
    dn4 = down_proj.reshape(NE, 2, HID // 2, EDIM)
    ws2d = w_sorted.reshape(RPAD, 1)
    hw_a = _mlp_half0(xs_a, gup4, dn4, ws2d, b2e, nbv)
    return _mlp_half1(xs_b, gup4, dn4, ws2d, b2e, nbv, hw_in=hw_a)


_SC_MESH = plsc.VectorSubcoreMesh(core_axis_name="c", subcore_axis_name="s")


def _sc_wid():
    return lax.axis_index("s") * SC_NC + lax.axis_index("c")


@functools.partial(
    pl.kernel,
    out_type=[jax.ShapeDtypeStruct((RPAD,), jnp.int32),
              jax.ShapeDtypeStruct((RPAD,), jnp.float32)],
    mesh=_SC_MESH,
    scratch_types=[pltpu.VMEM((2 * N_TOK,), jnp.int32),
                   pltpu.VMEM((2 * N_TOK,), jnp.float32),
                   pltpu.VMEM((RPAD,), jnp.int32),
                   pltpu.VMEM((RPAD,), jnp.float32)],
    compiler_params=pltpu.CompilerParams(needs_layout_passes=False),
)
def _sc_scatter_plan(pos_hbm, w_hbm, ts_hbm, ws_hbm, pos_v, w_v, ts_v, ws_v):
    @pl.when(_sc_wid() == 0)
    def _():
        pltpu.sync_copy(pos_hbm, pos_v)
        pltpu.sync_copy(w_hbm, w_v)

        def init(i, carry):
            ts_v[pl.ds(i * SC_L, SC_L)] = (
                (lax.iota(jnp.int32, SC_L) + i * SC_L) & (N_TOK - 1))
            ws_v[pl.ds(i * SC_L, SC_L)] = jnp.zeros((SC_L,), jnp.float32)
            return carry
        lax.fori_loop(0, RPAD // SC_L, init, 0)

        def scat(i, carry):
            sl = pl.ds(i * SC_L, SC_L)
            idx = pos_v[sl]
            tok = (lax.iota(jnp.int32, SC_L) + i * SC_L) & (N_TOK - 1)
            plsc.store_scatter(ts_v, [idx], tok)
            plsc.store_scatter(ws_v, [idx], w_v[sl])
            return carry
        lax.fori_loop(0, 2 * N_TOK // SC_L, scat, 0)
        pltpu.sync_copy(ts_v, ts_hbm)
        pltpu.sync_copy(ws_v, ws_hbm)


_GCH = 32
_GNB = 4


def _make_sc_gather_piece(row0, nrows, tag):
    gpt = nrows // SC_NW

    @functools.partial(
        pl.kernel,
        out_type=jax.ShapeDtypeStruct((nrows, HID), jnp.float32),
        mesh=_SC_MESH,
        scratch_types=[pltpu.VMEM((gpt,), jnp.int32),
                       [pltpu.VMEM((_GCH, HID), jnp.float32)] * _GNB,
                       [pltpu.SemaphoreType.DMA] * _GNB,
                       [pltpu.SemaphoreType.DMA] * _GNB],
        name=f"sc_gather_x_{tag}",
    )
    def _sc_gather_x(x_hbm, ts_hbm, xs_hbm, idx_v, rows_bufs, gsems, wsems):
        wid = _sc_wid()
        base = wid * gpt
        pltpu.sync_copy(ts_hbm.at[pl.ds(row0 + base, gpt)], idx_v)
        nch = gpt // _GCH
        nring = min(_GNB, nch)

        def issue_gather(i, b):
            pltpu.async_copy(x_hbm.at[idx_v.at[pl.ds(i * _GCH, _GCH)]],
                             rows_bufs[b], gsems[b])

        for b in range(nring):
            issue_gather(b, b)

        def step(i, carry):
            for b in range(_GNB):
                @pl.when(i % _GNB == b)
                def _():
                    pltpu.make_async_copy(
                        x_hbm.at[idx_v.at[pl.ds(0, _GCH)]], rows_bufs[b],
                        gsems[b]).wait()
                    pltpu.async_copy(rows_bufs[b],
                                     xs_hbm.at[pl.ds(base + i * _GCH, _GCH)],
                                     wsems[b])

                    @pl.when(i + _GNB < nch)
                    def _():
                        pltpu.make_async_copy(
                            rows_bufs[b], xs_hbm.at[pl.ds(base, _GCH)],
                            wsems[b]).wait()
                        issue_gather(i + _GNB, b)
            return carry
        lax.fori_loop(0, nch, step, 0)

        for b in range(nring):
            pltpu.make_async_copy(rows_bufs[b], xs_hbm.at[pl.ds(0, _GCH)],
                                  wsems[b]).wait()
    return _sc_gather_x


_sc_gather_half0 = _make_sc_gather_piece(0, _RSPLIT, 0)
_sc_gather_half1 = _make_sc_gather_piece(_RSPLIT, RPAD - _RSPLIT, 1)


_CCH = 32


@functools.partial(
    pl.kernel,
    out_type=jax.ShapeDtypeStruct((N_TOK, HID), jnp.float32),
    mesh=_SC_MESH,
    scratch_types=[pltpu.VMEM((_CCH,), jnp.int32),
                   pltpu.VMEM((_CCH,), jnp.int32),
                   pltpu.VMEM((_CCH, HID), jnp.float32),
                   pltpu.VMEM((_CCH, HID), jnp.float32),
                   pltpu.SemaphoreType.DMA,
                   pltpu.SemaphoreType.DMA],
)
def _sc_combine(hw_hbm, p0_hbm, p1_hbm, out_hbm, i0_v, i1_v, b0_v, b1_v,
                s0, s1):
    wid = _sc_wid()

    def chunk(i, carry):
        base = wid * (N_TOK // SC_NW) + i * _CCH
        pltpu.sync_copy(p0_hbm.at[pl.ds(base, _CCH)], i0_v)
        pltpu.sync_copy(p1_hbm.at[pl.ds(base, _CCH)], i1_v)
        c0 = pltpu.async_copy(hw_hbm.at[i0_v], b0_v, s0)
        c1 = pltpu.async_copy(hw_hbm.at[i1_v], b1_v, s1)
        c0.wait()
        c1.wait()

        def addrow(r, carry2):
            for cc in range(HID // SC_L):
                sl = pl.ds(cc * SC_L, SC_L)
                b0_v[r, sl] = b0_v[r, sl] + b1_v[r, sl]
            return carry2
        lax.fori_loop(0, _CCH, addrow, 0)
        pltpu.sync_copy(b0_v, out_hbm.at[pl.ds(base, _CCH)])
        return carry
    lax.fori_loop(0, N_TOK // SC_NW // _CCH, chunk, 0)


def kernel(x, router_scale, per_expert_scale, router_w, gate_up_proj,
           down_proj):
    pos2, w2, b2e, nb = _router_plan(x, router_scale, per_expert_scale,
                                     router_w)
    pos_flat = pos2.reshape(2 * N_TOK)
    w_flat = w2.reshape(2 * N_TOK)

    tok_sorted, w_sorted = _sc_scatter_plan(pos_flat, w_flat)
    xs_a = _sc_gather_half0(x, tok_sorted)
    xs_b = _sc_gather_half1(x, tok_sorted)
    hw = _grouped_mlp(xs_a, xs_b, gate_up_proj, down_proj, w_sorted,
                      b2e.reshape(NBLK), nb.reshape(1))
    out = _sc_combine(hw, pos2[0], pos2[1])
    return out
